# baseline (speedup 1.0000x reference)
import functools

import jax
import jax.numpy as jnp
from jax import lax
from jax.experimental import pallas as pl
from jax.experimental.pallas import tpu as pltpu
from jax.experimental.pallas import tpu_sc as plsc

EMB = 64
NC = 2
NS = 16
NW = NC * NS
C = 256
NB = 5
D = 2


@functools.partial(jax.jit, static_argnums=(2,))
def _gather_sc(idx, table, nblk):
    assert nblk % NB == 0
    ngroup = nblk // NB
    mesh = plsc.VectorSubcoreMesh(core_axis_name="c", subcore_axis_name="s")

    @functools.partial(
        pl.kernel,
        mesh=mesh,
        out_type=jax.ShapeDtypeStruct((NW, nblk, C, EMB), jnp.bfloat16),
        scratch_types=(
            [pltpu.VMEM((nblk, C), jnp.int32),
             pltpu.VMEM((NB, C, EMB), jnp.bfloat16)]
            + [pltpu.SemaphoreType.DMA] * (2 * NB)
        ),
        compiler_params=pltpu.CompilerParams(use_tc_tiling_on_sc=False),
    )
    def k(idx_hbm, table_hbm, out_hbm, idx_v, rows, *sems):
        gsem = sems[:NB]
        ssem = sems[NB:]
        wid = lax.axis_index("s") * NC + lax.axis_index("c")
        pltpu.sync_copy(idx_hbm.at[wid], idx_v)

        def fire_gather(j, b):
            pltpu.async_copy(table_hbm.at[idx_v.at[j]], rows.at[b], gsem[b])

        def wait_gather(b):
            pltpu.make_async_copy(out_hbm.at[wid, 0], rows.at[b],
                                  gsem[b]).wait()

        def fire_scatter(j, b):
            pltpu.async_copy(rows.at[b], out_hbm.at[wid, j], ssem[b])

        def wait_scatter(b):
            pltpu.make_async_copy(rows.at[b], out_hbm.at[wid, 0],
                                  ssem[b]).wait()

        for b in range(NB):
            fire_gather(b, b)
            if b >= D:
                b2 = b - D
                wait_gather(b2)
                fire_scatter(b2, b2)

        def group(g, carry):
            j0 = g * NB
            for b in range(NB):
                j = j0 + b
                wait_scatter(b)
                fire_gather(j, b)
                b2 = (b + NB - D) % NB
                wait_gather(b2)
                fire_scatter(j - D, b2)
            return carry

        lax.fori_loop(1, ngroup, group, 0)

        j0 = (ngroup - 1) * NB
        for b in range(NB - D, NB):
            wait_gather(b)
            fire_scatter(j0 + b, b)
        for b in range(NB):
            wait_scatter(b)

    return k(idx, table)


def kernel(char_ids, table):
    B, L = char_ids.shape
    total = B * L
    assert total % (NW * C) == 0
    nblk = total // (NW * C)
    idx = char_ids.reshape(NW, nblk, C)
    out = _gather_sc(idx, table.astype(jnp.bfloat16), nblk)
    return out.reshape(B, L, EMB).astype(jnp.float32)


# speedup vs baseline: 1.5220x; 1.5220x over previous
import functools

import jax
import jax.numpy as jnp
from jax import lax
from jax.experimental import pallas as pl
from jax.experimental.pallas import tpu as pltpu
from jax.experimental.pallas import tpu_sc as plsc

EMB = 64
NC = 2
NS = 16
NW = NC * NS
C = 512
NB = 2
D = 1


@functools.partial(jax.jit, static_argnums=(2,))
def _gather_sc(idx, table, nblk):
    assert nblk % NB == 0
    ngroup = nblk // NB
    mesh = plsc.VectorSubcoreMesh(core_axis_name="c", subcore_axis_name="s")

    @functools.partial(
        pl.kernel,
        mesh=mesh,
        out_type=jax.ShapeDtypeStruct((NW, nblk, C, EMB), jnp.float32),
        scratch_types=(
            [pltpu.VMEM((nblk, C), jnp.int32),
             pltpu.VMEM((NB, C, EMB), jnp.float32)]
            + [pltpu.SemaphoreType.DMA] * (2 * NB)
        ),
        compiler_params=pltpu.CompilerParams(use_tc_tiling_on_sc=False),
    )
    def k(idx_hbm, table_hbm, out_hbm, idx_v, rows, *sems):
        gsem = sems[:NB]
        ssem = sems[NB:]
        wid = lax.axis_index("s") * NC + lax.axis_index("c")
        pltpu.sync_copy(idx_hbm.at[wid], idx_v)

        def fire_gather(j, b):
            pltpu.async_copy(table_hbm.at[idx_v.at[j]], rows.at[b], gsem[b])

        def wait_gather(b):
            pltpu.make_async_copy(out_hbm.at[wid, 0], rows.at[b],
                                  gsem[b]).wait()

        def fire_scatter(j, b):
            pltpu.async_copy(rows.at[b], out_hbm.at[wid, j], ssem[b])

        def wait_scatter(b):
            pltpu.make_async_copy(rows.at[b], out_hbm.at[wid, 0],
                                  ssem[b]).wait()

        for b in range(NB):
            fire_gather(b, b)
            if b >= D:
                b2 = b - D
                wait_gather(b2)
                fire_scatter(b2, b2)

        def group(g, carry):
            j0 = g * NB
            for b in range(NB):
                j = j0 + b
                wait_scatter(b)
                fire_gather(j, b)
                b2 = (b + NB - D) % NB
                wait_gather(b2)
                fire_scatter(j - D, b2)
            return carry

        lax.fori_loop(1, ngroup, group, 0)

        j0 = (ngroup - 1) * NB
        for b in range(NB - D, NB):
            wait_gather(b)
            fire_scatter(j0 + b, b)
        for b in range(NB):
            wait_scatter(b)

    return k(idx, table)


def kernel(char_ids, table):
    B, L = char_ids.shape
    total = B * L
    assert total % (NW * C) == 0
    nblk = total // (NW * C)
    idx = char_ids.reshape(NW, nblk, C)
    out = _gather_sc(idx, table, nblk)
    return out.reshape(B, L, EMB)
